# baseline (speedup 1.0000x reference)
import functools

import jax
import jax.numpy as jnp
from jax import lax
from jax.experimental import pallas as pl
from jax.experimental.pallas import tpu as pltpu
from jax.experimental.pallas import tpu_sc as plsc

N_NODES = 10000
N_EDGES = 160000
D_IN = 256
D_OUT = 256

NW = 32
ROWS = 10240
OWN = ROWS // NW
SUB = 128
WROWS = 16
WAVE = WROWS * SUB
BATCH = 128
EDGES_PAD = 163840
IDX_ROWS = EDGES_PAD // SUB
NWAVE = EDGES_PAD // WAVE

_mesh = plsc.VectorSubcoreMesh(core_axis_name="c", subcore_axis_name="s")


@functools.partial(
    pl.kernel,
    out_type=[
        jax.ShapeDtypeStruct((ROWS, D_IN), jnp.float32),
        jax.ShapeDtypeStruct((ROWS,), jnp.float32),
    ],
    mesh=_mesh,
    compiler_params=pltpu.CompilerParams(needs_layout_passes=False),
    scratch_types=[
        pltpu.VMEM((WROWS, SUB), jnp.int32),
        pltpu.VMEM((WROWS, SUB), jnp.int32),
        pltpu.VMEM((WAVE + 16,), jnp.int32),
        pltpu.VMEM((WAVE + 16,), jnp.int32),
        pltpu.VMEM((BATCH, D_IN), jnp.float32),
        pltpu.VMEM((OWN, D_IN), jnp.float32),
        pltpu.VMEM((OWN,), jnp.float32),
        pltpu.SemaphoreType.DMA,
    ],
)
def _sc_aggregate(x_hbm, src_hbm, dst_hbm, sum_hbm, cnt_hbm,
                  srcv, dstv, srcC, dstC, gbuf, acc, cntv, sem):
    c = lax.axis_index("c")
    s = lax.axis_index("s")
    w = s * 2 + c
    lo = w * OWN

    zero16 = jnp.zeros((16,), jnp.float32)

    iota16 = lax.iota(jnp.int32, 16)
    lane0 = iota16 == 0
    one16f = jnp.ones((16,), jnp.float32)

    def _za(i, carry):
        for j in range(D_IN // 16):
            acc[i, pl.ds(j * 16, 16)] = zero16
        return carry
    lax.fori_loop(0, OWN, _za, 0)

    def _zc(i, carry):
        cntv[pl.ds(i * 16, 16)] = zero16
        return carry
    lax.fori_loop(0, OWN // 16, _zc, 0)

    def _wave(iw, carry):
        pltpu.sync_copy(src_hbm.at[pl.ds(iw * WROWS, WROWS)], srcv)
        pltpu.sync_copy(dst_hbm.at[pl.ds(iw * WROWS, WROWS)], dstv)

        def _pre(g, carry2):
            srcC[pl.ds(g * 16, 16)] = jnp.zeros((16,), jnp.int32)
            return carry2
        lax.fori_loop(0, (WAVE + 16) // 16, _pre, 0)

        def _scan(g, cur):
            r = g // (SUB // 16)
            k = g % (SUB // 16)
            sv = srcv[r, pl.ds(k * 16, 16)]
            dv = dstv[r, pl.ds(k * 16, 16)]
            loc = dv - lo
            inr = jnp.logical_and(loc >= 0, loc < OWN)
            inr32 = jnp.where(inr, jnp.int32(1), jnp.int32(0))
            pref = plsc.cumsum(inr32)
            pos16 = (cur + pref) - inr32
            plsc.store_scatter(srcC, [pos16], sv, mask=inr)
            plsc.store_scatter(dstC, [pos16], loc, mask=inr)
            return cur + jnp.sum(inr32)
        kk = lax.fori_loop(0, WAVE // 16, _scan, jnp.int32(0))

        def _batch(b, carry2):
            pltpu.async_copy(
                x_hbm.at[srcC.at[pl.ds(b * BATCH, BATCH)]],
                gbuf, sem).wait()
            kb = jnp.minimum(kk - b * BATCH, BATCH)

            def _edge(e, carry3):
                row16 = plsc.load_gather(
                    dstC, [jnp.broadcast_to(b * BATCH + e, (16,))])
                cv = plsc.load_gather(cntv, [row16])
                plsc.store_scatter(cntv, [row16], cv + one16f, mask=lane0)
                for cc in range(D_IN // 16):
                    colc = iota16 + cc * 16
                    av = plsc.load_gather(acc, [row16, colc])
                    gv = gbuf[e, pl.ds(cc * 16, 16)]
                    plsc.store_scatter(acc, [row16, colc], av + gv)
                return carry3
            lax.fori_loop(0, kb, _edge, 0)
            return carry2
        nb = (kk + BATCH - 1) // BATCH
        lax.fori_loop(0, nb, _batch, 0)
        return carry
    lax.fori_loop(0, NWAVE, _wave, 0)

    pltpu.sync_copy(acc, sum_hbm.at[pl.ds(lo, OWN)])
    pltpu.sync_copy(cntv, cnt_hbm.at[pl.ds(lo, OWN)])


def _finish_body(sum_ref, cnt_ref, x_ref, wl_ref, wr_ref, b_ref, o_ref):
    a = jnp.dot(sum_ref[...], wl_ref[...], preferred_element_type=jnp.float32)
    r = jnp.dot(x_ref[...], wr_ref[...], preferred_element_type=jnp.float32)
    inv = 1.0 / jnp.maximum(cnt_ref[...], 1.0)
    o_ref[...] = jnp.maximum(a * inv + r + b_ref[...], 0.0)


_BLK = 1000


def _tc_finish(summed, cnt2, x, wl, wr, b2):
    grid = (N_NODES // _BLK,)
    return pl.pallas_call(
        _finish_body,
        grid=grid,
        in_specs=[
            pl.BlockSpec((_BLK, D_IN), lambda i: (i, 0)),
            pl.BlockSpec((_BLK, 1), lambda i: (i, 0)),
            pl.BlockSpec((_BLK, D_IN), lambda i: (i, 0)),
            pl.BlockSpec((D_IN, D_OUT), lambda i: (0, 0)),
            pl.BlockSpec((D_IN, D_OUT), lambda i: (0, 0)),
            pl.BlockSpec((1, D_OUT), lambda i: (0, 0)),
        ],
        out_specs=pl.BlockSpec((_BLK, D_OUT), lambda i: (i, 0)),
        out_shape=jax.ShapeDtypeStruct((N_NODES, D_OUT), jnp.float32),
    )(summed, cnt2, x, wl, wr, b2)


def kernel(x, edge_index, W_l, b_l, W_r):
    x = x.astype(jnp.float32)
    src = edge_index[0].astype(jnp.int32)
    dst = edge_index[1].astype(jnp.int32)

    npad = EDGES_PAD - N_EDGES
    pad = jnp.arange(npad, dtype=jnp.int32)
    src_p = jnp.concatenate([src, pad % N_NODES])
    dst_p = jnp.concatenate([dst, N_NODES + pad % (ROWS - N_NODES)])
    src2d = src_p.reshape(IDX_ROWS, SUB)
    dst2d = dst_p.reshape(IDX_ROWS, SUB)

    summed, cnt = _sc_aggregate(x, src2d, dst2d)

    wl = W_l.T
    wr = W_r.T
    b2 = b_l.reshape(1, D_OUT)
    cnt2 = cnt.reshape(ROWS, 1)
    return _tc_finish(summed[:N_NODES], cnt2[:N_NODES], x, wl, wr, b2)


# speedup vs baseline: 6.6180x; 6.6180x over previous
import functools

import jax
import jax.numpy as jnp
from jax import lax
from jax.experimental import pallas as pl
from jax.experimental.pallas import tpu as pltpu
from jax.experimental.pallas import tpu_sc as plsc

N_NODES = 10000
N_EDGES = 160000
D_IN = 256
D_OUT = 256

NW = 32
ROWS = 10240
OWN = ROWS // NW
SUB = 128
WROWS = 16
WAVE = WROWS * SUB
BATCH = 128
EDGES_PAD = 163840
IDX_ROWS = EDGES_PAD // SUB
NWAVE = EDGES_PAD // WAVE

_mesh = plsc.VectorSubcoreMesh(core_axis_name="c", subcore_axis_name="s")


@functools.partial(
    pl.kernel,
    out_type=[
        jax.ShapeDtypeStruct((ROWS, D_IN), jnp.float32),
        jax.ShapeDtypeStruct((ROWS,), jnp.float32),
    ],
    mesh=_mesh,
    compiler_params=pltpu.CompilerParams(needs_layout_passes=False),
    scratch_types=[
        pltpu.VMEM((WROWS, SUB), jnp.int32),
        pltpu.VMEM((WROWS, SUB), jnp.int32),
        pltpu.VMEM((WAVE + 16,), jnp.int32),
        pltpu.VMEM((WAVE + 16,), jnp.int32),
        pltpu.VMEM((BATCH, D_IN // 2), jnp.int32),
        pltpu.VMEM((OWN, D_IN), jnp.float32),
        pltpu.VMEM((OWN,), jnp.float32),
        pltpu.SemaphoreType.DMA,
    ],
)
def _sc_aggregate(x_hbm, src_hbm, dst_hbm, sum_hbm, cnt_hbm,
                  srcv, dstv, srcC, dstC, gbuf, acc, cntv, sem):
    c = lax.axis_index("c")
    s = lax.axis_index("s")
    w = s * 2 + c
    lo = w * OWN

    zero16 = jnp.zeros((16,), jnp.float32)

    iota16 = lax.iota(jnp.int32, 16)
    lane0 = iota16 == 0
    one16f = jnp.ones((16,), jnp.float32)

    def _za(i, carry):
        for j in range(D_IN // 16):
            acc[i, pl.ds(j * 16, 16)] = zero16
        return carry
    lax.fori_loop(0, OWN, _za, 0)

    def _zc(i, carry):
        cntv[pl.ds(i * 16, 16)] = zero16
        return carry
    lax.fori_loop(0, OWN // 16, _zc, 0)

    def _wave(iw, carry):
        pltpu.sync_copy(src_hbm.at[pl.ds(iw * WROWS, WROWS)], srcv)
        pltpu.sync_copy(dst_hbm.at[pl.ds(iw * WROWS, WROWS)], dstv)

        def _pre(g, carry2):
            fill = (g * 16 + iota16 + w * 320) & 8191
            srcC[pl.ds(g * 16, 16)] = fill
            return carry2
        lax.fori_loop(0, (WAVE + 16) // 16, _pre, 0)

        def _scan(g, cur):
            r = g // (SUB // 16)
            k = g % (SUB // 16)
            sv = srcv[r, pl.ds(k * 16, 16)]
            dv = dstv[r, pl.ds(k * 16, 16)]
            loc = dv - lo
            inr = jnp.logical_and(loc >= 0, loc < OWN)
            inr32 = jnp.where(inr, jnp.int32(1), jnp.int32(0))
            pref = plsc.cumsum(inr32)
            pos16 = (cur + pref) - inr32
            plsc.store_scatter(srcC, [pos16], sv, mask=inr)
            plsc.store_scatter(dstC, [pos16], loc, mask=inr)
            return cur + jnp.sum(inr32)
        kk = lax.fori_loop(0, WAVE // 16, _scan, jnp.int32(0))

        def _batch(b, carry2):
            pltpu.async_copy(
                x_hbm.at[srcC.at[pl.ds(b * BATCH, BATCH)]],
                gbuf, sem).wait()
            kb = jnp.minimum(kk - b * BATCH, BATCH)

            def _edge(e, carry3):
                row16 = plsc.load_gather(
                    dstC, [jnp.broadcast_to(b * BATCH + e, (16,))])
                cv = plsc.load_gather(cntv, [row16])
                plsc.store_scatter(cntv, [row16], cv + one16f, mask=lane0)
                for cc in range(D_IN // 32):
                    ab32 = gbuf[e, pl.ds(cc * 16, 16)]
                    ab = plsc.bitcast(ab32, jnp.bfloat16)
                    ga, gb = plsc.unpack(ab,
                                         format=plsc.PackFormat.INTERLEAVED)
                    cola = cc * 32 + 2 * iota16
                    colb = cola + 1
                    ava = plsc.load_gather(acc, [row16, cola])
                    plsc.store_scatter(acc, [row16, cola], ava + ga)
                    avb = plsc.load_gather(acc, [row16, colb])
                    plsc.store_scatter(acc, [row16, colb], avb + gb)
                return carry3
            lax.fori_loop(0, kb, _edge, 0)
            return carry2
        nb = (kk + BATCH - 1) // BATCH
        lax.fori_loop(0, nb, _batch, 0)
        return carry
    lax.fori_loop(0, NWAVE, _wave, 0)

    pltpu.sync_copy(acc, sum_hbm.at[pl.ds(lo, OWN)])
    pltpu.sync_copy(cntv, cnt_hbm.at[pl.ds(lo, OWN)])


def _finish_body(sum_ref, cnt_ref, x_ref, wl_ref, wr_ref, b_ref, o_ref):
    a = jnp.dot(sum_ref[...], wl_ref[...], preferred_element_type=jnp.float32)
    r = jnp.dot(x_ref[...], wr_ref[...], preferred_element_type=jnp.float32)
    inv = 1.0 / jnp.maximum(cnt_ref[...], 1.0)
    o_ref[...] = jnp.maximum(a * inv + r + b_ref[...], 0.0)


_BLK = 1000


def _tc_finish(summed, cnt2, x, wl, wr, b2):
    grid = (N_NODES // _BLK,)
    return pl.pallas_call(
        _finish_body,
        grid=grid,
        in_specs=[
            pl.BlockSpec((_BLK, D_IN), lambda i: (i, 0)),
            pl.BlockSpec((_BLK, 1), lambda i: (i, 0)),
            pl.BlockSpec((_BLK, D_IN), lambda i: (i, 0)),
            pl.BlockSpec((D_IN, D_OUT), lambda i: (0, 0)),
            pl.BlockSpec((D_IN, D_OUT), lambda i: (0, 0)),
            pl.BlockSpec((1, D_OUT), lambda i: (0, 0)),
        ],
        out_specs=pl.BlockSpec((_BLK, D_OUT), lambda i: (i, 0)),
        out_shape=jax.ShapeDtypeStruct((N_NODES, D_OUT), jnp.float32),
    )(summed, cnt2, x, wl, wr, b2)


def kernel(x, edge_index, W_l, b_l, W_r):
    x = x.astype(jnp.float32)
    src = edge_index[0].astype(jnp.int32)
    dst = edge_index[1].astype(jnp.int32)

    npad = EDGES_PAD - N_EDGES
    pad = jnp.arange(npad, dtype=jnp.int32)
    src_p = jnp.concatenate([src, pad % N_NODES])
    dst_p = jnp.concatenate([dst, N_NODES + pad % (ROWS - N_NODES)])
    src2d = src_p.reshape(IDX_ROWS, SUB)
    dst2d = dst_p.reshape(IDX_ROWS, SUB)

    xbf = x.astype(jnp.bfloat16).reshape(N_NODES, D_IN // 2, 2)
    xw = jax.lax.bitcast_convert_type(xbf, jnp.int32)
    summed, cnt = _sc_aggregate(xw, src2d, dst2d)

    wl = W_l.T
    wr = W_r.T
    b2 = b_l.reshape(1, D_OUT)
    cnt2 = cnt.reshape(ROWS, 1)
    return _tc_finish(summed[:N_NODES], cnt2[:N_NODES], x, wl, wr, b2)


# speedup vs baseline: 7.8098x; 1.1801x over previous
import functools

import jax
import jax.numpy as jnp
from jax import lax
from jax.experimental import pallas as pl
from jax.experimental.pallas import tpu as pltpu
from jax.experimental.pallas import tpu_sc as plsc

N_NODES = 10000
N_EDGES = 160000
D_IN = 256
D_OUT = 256

NW = 32
ROWS = 10240
OWN = ROWS // NW
SUB = 128
WROWS = 16
WAVE = WROWS * SUB
BATCH = 128
EDGES_PAD = 163840
IDX_ROWS = EDGES_PAD // SUB
NWAVE = EDGES_PAD // WAVE

_mesh = plsc.VectorSubcoreMesh(core_axis_name="c", subcore_axis_name="s")


@functools.partial(
    pl.kernel,
    out_type=[
        jax.ShapeDtypeStruct((ROWS, D_IN), jnp.float32),
        jax.ShapeDtypeStruct((ROWS,), jnp.float32),
    ],
    mesh=_mesh,
    compiler_params=pltpu.CompilerParams(needs_layout_passes=False),
    scratch_types=[
        pltpu.VMEM((WROWS, SUB), jnp.int32),
        pltpu.VMEM((WROWS, SUB), jnp.int32),
        pltpu.VMEM((WAVE + 16,), jnp.int32),
        pltpu.VMEM((WAVE + 16,), jnp.int32),
        pltpu.VMEM((WAVE + 16,), jnp.int32),
        pltpu.VMEM((WAVE + 16,), jnp.int32),
        pltpu.VMEM((BATCH, D_IN), jnp.float32),
        pltpu.VMEM((OWN, D_IN), jnp.float32),
        pltpu.VMEM((OWN,), jnp.float32),
        pltpu.SemaphoreType.DMA,
    ],
)
def _sc_aggregate(x_hbm, src_hbm, dst_hbm, sum_hbm, cnt_hbm,
                  srcv, dstv, srcCA, dstCA, srcCB, dstCB, gbuf, acc, cntv, sem):
    c = lax.axis_index("c")
    s = lax.axis_index("s")
    w = s * 2 + c
    lo = w * OWN

    zero16 = jnp.zeros((16,), jnp.float32)

    iota16 = lax.iota(jnp.int32, 16)
    lane0 = iota16 == 0
    one16f = jnp.ones((16,), jnp.float32)

    def _za(i, carry):
        for j in range(D_IN // 16):
            acc[i, pl.ds(j * 16, 16)] = zero16
        return carry
    lax.fori_loop(0, OWN, _za, 0)

    def _zc(i, carry):
        cntv[pl.ds(i * 16, 16)] = zero16
        return carry
    lax.fori_loop(0, OWN // 16, _zc, 0)

    def _scan_wave(iw, sC, dC):
        pltpu.sync_copy(src_hbm.at[pl.ds(iw * WROWS, WROWS)], srcv)
        pltpu.sync_copy(dst_hbm.at[pl.ds(iw * WROWS, WROWS)], dstv)

        def _pre(g, carry2):
            fill = (g * 16 + iota16 + w * 320) & 8191
            sC[pl.ds(g * 16, 16)] = fill
            return carry2
        lax.fori_loop(0, (WAVE + 16) // 16, _pre, 0)

        def _scan(g, cur):
            r = g // (SUB // 16)
            k = g % (SUB // 16)
            sv = srcv[r, pl.ds(k * 16, 16)]
            dv = dstv[r, pl.ds(k * 16, 16)]
            loc = dv - lo
            inr = jnp.logical_and(loc >= 0, loc < OWN)
            inr32 = jnp.where(inr, jnp.int32(1), jnp.int32(0))
            pref = plsc.cumsum(inr32)
            pos16 = (cur + pref) - inr32
            plsc.store_scatter(sC, [pos16], sv, mask=inr)
            plsc.store_scatter(dC, [pos16], loc, mask=inr)
            return cur + jnp.sum(inr32)
        return lax.fori_loop(0, WAVE // 16, _scan, jnp.int32(0))

    def _fire(sC):
        pltpu.async_copy(x_hbm.at[sC.at[pl.ds(0, BATCH)]], gbuf, sem)

    def _edges(dC, start, kb):
        def _edge(e, carry3):
            row16 = plsc.load_gather(
                dC, [jnp.broadcast_to(start + e, (16,))])
            cv = plsc.load_gather(cntv, [row16])
            plsc.store_scatter(cntv, [row16], cv + one16f, mask=lane0)
            for cc in range(D_IN // 16):
                colc = iota16 + cc * 16
                av = plsc.load_gather(acc, [row16, colc])
                gv = gbuf[e, pl.ds(cc * 16, 16)]
                plsc.store_scatter(acc, [row16, colc], av + gv)
            return carry3
        lax.fori_loop(0, kb, _edge, 0)

    def _rmw(sC, dC, k):
        pltpu.make_async_copy(x_hbm.at[sC.at[pl.ds(0, BATCH)]],
                              gbuf, sem).wait()
        _edges(dC, 0, jnp.minimum(k, BATCH))

        def _batch(b, carry2):
            pltpu.async_copy(x_hbm.at[sC.at[pl.ds(b * BATCH, BATCH)]],
                             gbuf, sem).wait()
            _edges(dC, b * BATCH, jnp.minimum(k - b * BATCH, BATCH))
            return carry2
        nb = (k + BATCH - 1) // BATCH
        lax.fori_loop(1, nb, _batch, 0)

    kA = _scan_wave(0, srcCA, dstCA)
    _fire(srcCA)

    def _pair(it, kA):
        kB = _scan_wave(2 * it + 1, srcCB, dstCB)
        _rmw(srcCA, dstCA, kA)
        _fire(srcCB)
        kA2 = _scan_wave(2 * it + 2, srcCA, dstCA)
        _rmw(srcCB, dstCB, kB)
        _fire(srcCA)
        return kA2
    kA = lax.fori_loop(0, (NWAVE - 2) // 2, _pair, kA)
    kB = _scan_wave(NWAVE - 1, srcCB, dstCB)
    _rmw(srcCA, dstCA, kA)
    _fire(srcCB)
    _rmw(srcCB, dstCB, kB)

    pltpu.sync_copy(acc, sum_hbm.at[pl.ds(lo, OWN)])
    pltpu.sync_copy(cntv, cnt_hbm.at[pl.ds(lo, OWN)])


def _finish_body(sum_ref, cnt_ref, x_ref, wl_ref, wr_ref, b_ref, o_ref):
    a = jnp.dot(sum_ref[...], wl_ref[...], preferred_element_type=jnp.float32)
    r = jnp.dot(x_ref[...], wr_ref[...], preferred_element_type=jnp.float32)
    inv = 1.0 / jnp.maximum(cnt_ref[...], 1.0)
    o_ref[...] = jnp.maximum(a * inv + r + b_ref[...], 0.0)


_BLK = 1000


def _tc_finish(summed, cnt2, x, wl, wr, b2):
    grid = (N_NODES // _BLK,)
    return pl.pallas_call(
        _finish_body,
        grid=grid,
        in_specs=[
            pl.BlockSpec((_BLK, D_IN), lambda i: (i, 0)),
            pl.BlockSpec((_BLK, 1), lambda i: (i, 0)),
            pl.BlockSpec((_BLK, D_IN), lambda i: (i, 0)),
            pl.BlockSpec((D_IN, D_OUT), lambda i: (0, 0)),
            pl.BlockSpec((D_IN, D_OUT), lambda i: (0, 0)),
            pl.BlockSpec((1, D_OUT), lambda i: (0, 0)),
        ],
        out_specs=pl.BlockSpec((_BLK, D_OUT), lambda i: (i, 0)),
        out_shape=jax.ShapeDtypeStruct((N_NODES, D_OUT), jnp.float32),
    )(summed, cnt2, x, wl, wr, b2)


def kernel(x, edge_index, W_l, b_l, W_r):
    x = x.astype(jnp.float32)
    src = edge_index[0].astype(jnp.int32)
    dst = edge_index[1].astype(jnp.int32)

    npad = EDGES_PAD - N_EDGES
    pad = jnp.arange(npad, dtype=jnp.int32)
    src_p = jnp.concatenate([src, pad % N_NODES])
    dst_p = jnp.concatenate([dst, N_NODES + pad % (ROWS - N_NODES)])
    src2d = src_p.reshape(IDX_ROWS, SUB)
    dst2d = dst_p.reshape(IDX_ROWS, SUB)

    summed, cnt = _sc_aggregate(x, src2d, dst2d)

    wl = W_l.T
    wr = W_r.T
    b2 = b_l.reshape(1, D_OUT)
    cnt2 = cnt.reshape(ROWS, 1)
    return _tc_finish(summed[:N_NODES], cnt2[:N_NODES], x, wl, wr, b2)


# speedup vs baseline: 8.0526x; 1.0311x over previous
import functools

import jax
import jax.numpy as jnp
from jax import lax
from jax.experimental import pallas as pl
from jax.experimental.pallas import tpu as pltpu
from jax.experimental.pallas import tpu_sc as plsc

N_NODES = 10000
N_EDGES = 160000
D_IN = 256
D_OUT = 256

NW = 32
ROWS = 10240
OWN = ROWS // NW
SUB = 128
WROWS = 16
WAVE = WROWS * SUB
BATCH = 128
EDGES_PAD = 163840
IDX_ROWS = EDGES_PAD // SUB
NWAVE = EDGES_PAD // WAVE

_mesh = plsc.VectorSubcoreMesh(core_axis_name="c", subcore_axis_name="s")


@functools.partial(
    pl.kernel,
    out_type=[
        jax.ShapeDtypeStruct((ROWS, D_IN), jnp.float32),
        jax.ShapeDtypeStruct((ROWS,), jnp.float32),
    ],
    mesh=_mesh,
    compiler_params=pltpu.CompilerParams(needs_layout_passes=False),
    scratch_types=[
        pltpu.VMEM((WROWS, SUB), jnp.int32),
        pltpu.VMEM((WROWS, SUB), jnp.int32),
        pltpu.VMEM((WAVE + 16,), jnp.int32),
        pltpu.VMEM((WAVE + 16,), jnp.int32),
        pltpu.VMEM((WAVE + 16,), jnp.int32),
        pltpu.VMEM((WAVE + 16,), jnp.int32),
        pltpu.VMEM((BATCH, D_IN), jnp.float32),
        pltpu.VMEM((OWN, D_IN), jnp.float32),
        pltpu.VMEM((OWN,), jnp.float32),
        pltpu.SemaphoreType.DMA,
    ],
)
def _sc_aggregate(x_hbm, src_hbm, dst_hbm, sum_hbm, cnt_hbm,
                  srcv, dstv, srcCA, dstCA, srcCB, dstCB, gbuf, acc, cntv, sem):
    c = lax.axis_index("c")
    s = lax.axis_index("s")
    w = s * 2 + c
    lo = w * OWN

    zero16 = jnp.zeros((16,), jnp.float32)

    iota16 = lax.iota(jnp.int32, 16)
    lane0 = iota16 == 0
    one16f = jnp.ones((16,), jnp.float32)

    def _za(i, carry):
        for j in range(D_IN // 16):
            acc[i, pl.ds(j * 16, 16)] = zero16
        return carry
    lax.fori_loop(0, OWN, _za, 0)

    def _zc(i, carry):
        cntv[pl.ds(i * 16, 16)] = zero16
        return carry
    lax.fori_loop(0, OWN // 16, _zc, 0)

    HB = WAVE // 2
    GB = BATCH // 2

    def _scan_wave(iw, sC, dC):
        pltpu.sync_copy(src_hbm.at[pl.ds(iw * WROWS, WROWS)], srcv)
        pltpu.sync_copy(dst_hbm.at[pl.ds(iw * WROWS, WROWS)], dstv)

        def _pre(g, carry2):
            fill = (g * 16 + iota16 + w * 320) & 8191
            sC[pl.ds(g * 16, 16)] = fill
            return carry2
        lax.fori_loop(0, (WAVE + 16) // 16, _pre, 0)

        def _scan(g, carry):
            curA, curB = carry
            rA, kA_ = g // (SUB // 16), g % (SUB // 16)
            rB, kB_ = (g + WAVE // 32) // (SUB // 16), g % (SUB // 16)
            svA = srcv[rA, pl.ds(kA_ * 16, 16)]
            dvA = dstv[rA, pl.ds(kA_ * 16, 16)]
            svB = srcv[rB, pl.ds(kB_ * 16, 16)]
            dvB = dstv[rB, pl.ds(kB_ * 16, 16)]
            locA = dvA - lo
            locB = dvB - lo
            inrA = jnp.logical_and(locA >= 0, locA < OWN)
            inrB = jnp.logical_and(locB >= 0, locB < OWN)
            iA = jnp.where(inrA, jnp.int32(1), jnp.int32(0))
            iB = jnp.where(inrB, jnp.int32(1), jnp.int32(0))
            prefA = plsc.cumsum(iA)
            prefB = plsc.cumsum(iB)
            posA = (curA + prefA) - iA
            posB = (HB + curB + prefB) - iB
            plsc.store_scatter(sC, [posA], svA, mask=inrA)
            plsc.store_scatter(dC, [posA], locA, mask=inrA)
            plsc.store_scatter(sC, [posB], svB, mask=inrB)
            plsc.store_scatter(dC, [posB], locB, mask=inrB)
            return (curA + jnp.sum(iA), curB + jnp.sum(iB))
        return lax.fori_loop(0, WAVE // 32, _scan,
                             (jnp.int32(0), jnp.int32(0)))

    def _fire(sC):
        pltpu.async_copy(x_hbm.at[sC.at[pl.ds(0, GB)]],
                         gbuf.at[pl.ds(0, GB)], sem)
        pltpu.async_copy(x_hbm.at[sC.at[pl.ds(HB, GB)]],
                         gbuf.at[pl.ds(GB, GB)], sem)

    def _edges(dC, start, gbase, kb):
        def _edge(e, carry3):
            row16 = plsc.load_gather(
                dC, [jnp.broadcast_to(start + e, (16,))])
            cv = plsc.load_gather(cntv, [row16])
            plsc.store_scatter(cntv, [row16], cv + one16f, mask=lane0)
            for cc in range(D_IN // 16):
                colc = iota16 + cc * 16
                av = plsc.load_gather(acc, [row16, colc])
                gv = gbuf[gbase + e, pl.ds(cc * 16, 16)]
                plsc.store_scatter(acc, [row16, colc], av + gv)
            return carry3
        lax.fori_loop(0, kb, _edge, 0)

    def _rmw(sC, dC, ks):
        kA, kB = ks
        pltpu.make_async_copy(x_hbm.at[sC.at[pl.ds(0, GB)]],
                              gbuf.at[pl.ds(0, GB)], sem).wait()
        pltpu.make_async_copy(x_hbm.at[sC.at[pl.ds(HB, GB)]],
                              gbuf.at[pl.ds(GB, GB)], sem).wait()
        _edges(dC, 0, 0, jnp.minimum(kA, GB))
        _edges(dC, HB, GB, jnp.minimum(kB, GB))

        def _batchA(b, carry2):
            pltpu.async_copy(x_hbm.at[sC.at[pl.ds(b * GB, GB)]],
                             gbuf.at[pl.ds(0, GB)], sem).wait()
            _edges(dC, b * GB, 0, jnp.minimum(kA - b * GB, GB))
            return carry2
        lax.fori_loop(1, (kA + GB - 1) // GB, _batchA, 0)

        def _batchB(b, carry2):
            pltpu.async_copy(x_hbm.at[sC.at[pl.ds(HB + b * GB, GB)]],
                             gbuf.at[pl.ds(GB, GB)], sem).wait()
            _edges(dC, HB + b * GB, GB, jnp.minimum(kB - b * GB, GB))
            return carry2
        lax.fori_loop(1, (kB + GB - 1) // GB, _batchB, 0)

    kA = _scan_wave(0, srcCA, dstCA)
    _fire(srcCA)

    def _pair(it, kA):
        kB = _scan_wave(2 * it + 1, srcCB, dstCB)
        _rmw(srcCA, dstCA, kA)
        _fire(srcCB)
        kA2 = _scan_wave(2 * it + 2, srcCA, dstCA)
        _rmw(srcCB, dstCB, kB)
        _fire(srcCA)
        return kA2
    kA = lax.fori_loop(0, (NWAVE - 2) // 2, _pair, kA)
    kB = _scan_wave(NWAVE - 1, srcCB, dstCB)
    _rmw(srcCA, dstCA, kA)
    _fire(srcCB)
    _rmw(srcCB, dstCB, kB)

    pltpu.sync_copy(acc, sum_hbm.at[pl.ds(lo, OWN)])
    pltpu.sync_copy(cntv, cnt_hbm.at[pl.ds(lo, OWN)])


def _finish_body(sum_ref, cnt_ref, x_ref, wl_ref, wr_ref, b_ref, o_ref):
    a = jnp.dot(sum_ref[...], wl_ref[...], preferred_element_type=jnp.float32)
    r = jnp.dot(x_ref[...], wr_ref[...], preferred_element_type=jnp.float32)
    inv = 1.0 / jnp.maximum(cnt_ref[...], 1.0)
    o_ref[...] = jnp.maximum(a * inv + r + b_ref[...], 0.0)


_BLK = 1000


def _tc_finish(summed, cnt2, x, wl, wr, b2):
    grid = (N_NODES // _BLK,)
    return pl.pallas_call(
        _finish_body,
        grid=grid,
        in_specs=[
            pl.BlockSpec((_BLK, D_IN), lambda i: (i, 0)),
            pl.BlockSpec((_BLK, 1), lambda i: (i, 0)),
            pl.BlockSpec((_BLK, D_IN), lambda i: (i, 0)),
            pl.BlockSpec((D_IN, D_OUT), lambda i: (0, 0)),
            pl.BlockSpec((D_IN, D_OUT), lambda i: (0, 0)),
            pl.BlockSpec((1, D_OUT), lambda i: (0, 0)),
        ],
        out_specs=pl.BlockSpec((_BLK, D_OUT), lambda i: (i, 0)),
        out_shape=jax.ShapeDtypeStruct((N_NODES, D_OUT), jnp.float32),
    )(summed, cnt2, x, wl, wr, b2)


def kernel(x, edge_index, W_l, b_l, W_r):
    x = x.astype(jnp.float32)
    src = edge_index[0].astype(jnp.int32)
    dst = edge_index[1].astype(jnp.int32)

    npad = EDGES_PAD - N_EDGES
    pad = jnp.arange(npad, dtype=jnp.int32)
    src_p = jnp.concatenate([src, pad % N_NODES])
    dst_p = jnp.concatenate([dst, N_NODES + pad % (ROWS - N_NODES)])
    src2d = src_p.reshape(IDX_ROWS, SUB)
    dst2d = dst_p.reshape(IDX_ROWS, SUB)

    summed, cnt = _sc_aggregate(x, src2d, dst2d)

    wl = W_l.T
    wr = W_r.T
    b2 = b_l.reshape(1, D_OUT)
    cnt2 = cnt.reshape(ROWS, 1)
    return _tc_finish(summed[:N_NODES], cnt2[:N_NODES], x, wl, wr, b2)


# speedup vs baseline: 8.3309x; 1.0346x over previous
import functools

import jax
import jax.numpy as jnp
from jax import lax
from jax.experimental import pallas as pl
from jax.experimental.pallas import tpu as pltpu
from jax.experimental.pallas import tpu_sc as plsc

N_NODES = 10000
N_EDGES = 160000
D_IN = 256
D_OUT = 256

NW = 32
ROWS = 10240
OWN = ROWS // NW
SUB = 128
WROWS = 16
WAVE = WROWS * SUB
BATCH = 128
EDGES_PAD = 163840
IDX_ROWS = EDGES_PAD // SUB
NWAVE = EDGES_PAD // WAVE

_mesh = plsc.VectorSubcoreMesh(core_axis_name="c", subcore_axis_name="s")


@functools.partial(
    pl.kernel,
    out_type=[
        jax.ShapeDtypeStruct((ROWS, D_IN), jnp.float32),
        jax.ShapeDtypeStruct((ROWS,), jnp.float32),
    ],
    mesh=_mesh,
    compiler_params=pltpu.CompilerParams(needs_layout_passes=False),
    scratch_types=[
        pltpu.VMEM((WROWS, SUB), jnp.int32),
        pltpu.VMEM((WROWS, SUB), jnp.int32),
        pltpu.VMEM((WAVE + 16,), jnp.int32),
        pltpu.VMEM((WAVE + 16,), jnp.int32),
        pltpu.VMEM((WAVE + 16,), jnp.int32),
        pltpu.VMEM((WAVE + 16,), jnp.int32),
        pltpu.VMEM((BATCH, D_IN), jnp.float32),
        pltpu.VMEM((OWN, D_IN), jnp.float32),
        pltpu.VMEM((OWN,), jnp.float32),
        pltpu.SemaphoreType.DMA,
    ],
)
def _sc_aggregate(x_hbm, src_hbm, dst_hbm, sum_hbm, cnt_hbm,
                  srcv, dstv, srcCA, dstCA, srcCB, dstCB, gbuf, acc, cntv, sem):
    c = lax.axis_index("c")
    s = lax.axis_index("s")
    w = s * 2 + c
    lo = w * OWN

    zero16 = jnp.zeros((16,), jnp.float32)

    iota16 = lax.iota(jnp.int32, 16)
    lane0 = iota16 == 0
    one16f = jnp.ones((16,), jnp.float32)

    def _za(i, carry):
        for j in range(D_IN // 16):
            acc[i, pl.ds(j * 16, 16)] = zero16
        return carry
    lax.fori_loop(0, OWN, _za, 0)

    def _zc(i, carry):
        cntv[pl.ds(i * 16, 16)] = zero16
        return carry
    lax.fori_loop(0, OWN // 16, _zc, 0)

    HB = WAVE // 2
    GB = BATCH // 2

    def _prefill(sC, lo_g, hi_g):
        def _pre(g, carry2):
            fill = (g * 16 + iota16 + w * 320) & 8191
            sC[pl.ds(g * 16, 16)] = fill
            return carry2
        lax.fori_loop(lo_g, hi_g, _pre, 0)

    _prefill(srcCA, 0, (WAVE + 16) // 16)
    _prefill(srcCB, 0, (WAVE + 16) // 16)

    def _scan_wave(iw, sC, dC):
        pltpu.sync_copy(src_hbm.at[pl.ds(iw * WROWS, WROWS)], srcv)
        pltpu.sync_copy(dst_hbm.at[pl.ds(iw * WROWS, WROWS)], dstv)

        def _scan(g, carry):
            curA, curB = carry
            rA, kA_ = g // (SUB // 16), g % (SUB // 16)
            rB, kB_ = (g + WAVE // 32) // (SUB // 16), g % (SUB // 16)
            svA = srcv[rA, pl.ds(kA_ * 16, 16)]
            dvA = dstv[rA, pl.ds(kA_ * 16, 16)]
            svB = srcv[rB, pl.ds(kB_ * 16, 16)]
            dvB = dstv[rB, pl.ds(kB_ * 16, 16)]
            locA = dvA - lo
            locB = dvB - lo
            inrA = jnp.logical_and(locA >= 0, locA < OWN)
            inrB = jnp.logical_and(locB >= 0, locB < OWN)
            iA = jnp.where(inrA, jnp.int32(1), jnp.int32(0))
            iB = jnp.where(inrB, jnp.int32(1), jnp.int32(0))
            prefA = plsc.cumsum(iA)
            prefB = plsc.cumsum(iB)
            posA = (curA + prefA) - iA
            posB = (HB + curB + prefB) - iB
            plsc.store_scatter(sC, [posA], svA, mask=inrA)
            plsc.store_scatter(dC, [posA], locA, mask=inrA)
            plsc.store_scatter(sC, [posB], svB, mask=inrB)
            plsc.store_scatter(dC, [posB], locB, mask=inrB)
            return (curA + jnp.sum(iA), curB + jnp.sum(iB))
        return lax.fori_loop(0, WAVE // 32, _scan,
                             (jnp.int32(0), jnp.int32(0)))

    def _fire(sC):
        pltpu.async_copy(x_hbm.at[sC.at[pl.ds(0, GB)]],
                         gbuf.at[pl.ds(0, GB)], sem)
        pltpu.async_copy(x_hbm.at[sC.at[pl.ds(HB, GB)]],
                         gbuf.at[pl.ds(GB, GB)], sem)

    def _edges(dC, start, gbase, kb):
        def _edge(e, carry3):
            row16 = plsc.load_gather(
                dC, [jnp.broadcast_to(start + e, (16,))])
            cv = plsc.load_gather(cntv, [row16])
            plsc.store_scatter(cntv, [row16], cv + one16f, mask=lane0)
            for cc in range(D_IN // 16):
                colc = iota16 + cc * 16
                av = plsc.load_gather(acc, [row16, colc])
                gv = gbuf[gbase + e, pl.ds(cc * 16, 16)]
                plsc.store_scatter(acc, [row16, colc], av + gv)
            return carry3
        lax.fori_loop(0, kb, _edge, 0)

    def _rmw(sC, dC, ks):
        kA, kB = ks
        pltpu.make_async_copy(x_hbm.at[sC.at[pl.ds(0, GB)]],
                              gbuf.at[pl.ds(0, GB)], sem).wait()
        pltpu.make_async_copy(x_hbm.at[sC.at[pl.ds(HB, GB)]],
                              gbuf.at[pl.ds(GB, GB)], sem).wait()
        _edges(dC, 0, 0, jnp.minimum(kA, GB))
        _edges(dC, HB, GB, jnp.minimum(kB, GB))

        def _batchA(b, carry2):
            pltpu.async_copy(x_hbm.at[sC.at[pl.ds(b * GB, GB)]],
                             gbuf.at[pl.ds(0, GB)], sem).wait()
            _edges(dC, b * GB, 0, jnp.minimum(kA - b * GB, GB))
            return carry2
        lax.fori_loop(1, (kA + GB - 1) // GB, _batchA, 0)

        def _batchB(b, carry2):
            pltpu.async_copy(x_hbm.at[sC.at[pl.ds(HB + b * GB, GB)]],
                             gbuf.at[pl.ds(GB, GB)], sem).wait()
            _edges(dC, HB + b * GB, GB, jnp.minimum(kB - b * GB, GB))
            return carry2
        lax.fori_loop(1, (kB + GB - 1) // GB, _batchB, 0)
        _prefill(sC, 0, (kA + 15) // 16)
        _prefill(sC, HB // 16, (HB + kB + 15) // 16)

    kA = _scan_wave(0, srcCA, dstCA)
    _fire(srcCA)

    def _pair(it, kA):
        kB = _scan_wave(2 * it + 1, srcCB, dstCB)
        _rmw(srcCA, dstCA, kA)
        _fire(srcCB)
        kA2 = _scan_wave(2 * it + 2, srcCA, dstCA)
        _rmw(srcCB, dstCB, kB)
        _fire(srcCA)
        return kA2
    kA = lax.fori_loop(0, (NWAVE - 2) // 2, _pair, kA)
    kB = _scan_wave(NWAVE - 1, srcCB, dstCB)
    _rmw(srcCA, dstCA, kA)
    _fire(srcCB)
    _rmw(srcCB, dstCB, kB)

    pltpu.sync_copy(acc, sum_hbm.at[pl.ds(lo, OWN)])
    pltpu.sync_copy(cntv, cnt_hbm.at[pl.ds(lo, OWN)])


def _finish_body(sum_ref, cnt_ref, x_ref, wl_ref, wr_ref, b_ref, o_ref):
    a = jnp.dot(sum_ref[...], wl_ref[...], preferred_element_type=jnp.float32)
    r = jnp.dot(x_ref[...], wr_ref[...], preferred_element_type=jnp.float32)
    inv = 1.0 / jnp.maximum(cnt_ref[...], 1.0)
    o_ref[...] = jnp.maximum(a * inv + r + b_ref[...], 0.0)


_BLK = 1000


def _tc_finish(summed, cnt2, x, wl, wr, b2):
    grid = (N_NODES // _BLK,)
    return pl.pallas_call(
        _finish_body,
        grid=grid,
        in_specs=[
            pl.BlockSpec((_BLK, D_IN), lambda i: (i, 0)),
            pl.BlockSpec((_BLK, 1), lambda i: (i, 0)),
            pl.BlockSpec((_BLK, D_IN), lambda i: (i, 0)),
            pl.BlockSpec((D_IN, D_OUT), lambda i: (0, 0)),
            pl.BlockSpec((D_IN, D_OUT), lambda i: (0, 0)),
            pl.BlockSpec((1, D_OUT), lambda i: (0, 0)),
        ],
        out_specs=pl.BlockSpec((_BLK, D_OUT), lambda i: (i, 0)),
        out_shape=jax.ShapeDtypeStruct((N_NODES, D_OUT), jnp.float32),
    )(summed, cnt2, x, wl, wr, b2)


def kernel(x, edge_index, W_l, b_l, W_r):
    x = x.astype(jnp.float32)
    src = edge_index[0].astype(jnp.int32)
    dst = edge_index[1].astype(jnp.int32)

    npad = EDGES_PAD - N_EDGES
    pad = jnp.arange(npad, dtype=jnp.int32)
    src_p = jnp.concatenate([src, pad % N_NODES])
    dst_p = jnp.concatenate([dst, N_NODES + pad % (ROWS - N_NODES)])
    src2d = src_p.reshape(IDX_ROWS, SUB)
    dst2d = dst_p.reshape(IDX_ROWS, SUB)

    summed, cnt = _sc_aggregate(x, src2d, dst2d)

    wl = W_l.T
    wr = W_r.T
    b2 = b_l.reshape(1, D_OUT)
    cnt2 = cnt.reshape(ROWS, 1)
    return _tc_finish(summed[:N_NODES], cnt2[:N_NODES], x, wl, wr, b2)
